# Initial kernel scaffold; baseline (speedup 1.0000x reference)
import jax
import jax.numpy as jnp
from jax.experimental import pallas as pl


def kernel(x, emb_wi):
    raise NotImplementedError("write your pallas kernel here")


# speedup vs baseline: 4.0859x; 4.0859x over previous
import functools

import jax
import jax.numpy as jnp
from jax import lax
from jax.experimental import pallas as pl
from jax.experimental.pallas import tpu as pltpu
from jax.experimental.pallas import tpu_sc as plsc

_CH = 128


@functools.lru_cache(maxsize=None)
def _build_gather(n_rows: int, vocab: int, dim: int):
    info = plsc.get_sparse_core_info()
    nc, ns = info.num_cores, info.num_subcores
    nw = nc * ns
    assert n_rows % (nw * _CH) == 0
    b_per_w = n_rows // nw
    n_ch = b_per_w // _CH

    mesh = plsc.VectorSubcoreMesh(core_axis_name="c", subcore_axis_name="s")

    @functools.partial(
        pl.kernel,
        mesh=mesh,
        compiler_params=pltpu.CompilerParams(use_tc_tiling_on_sc=False),
        out_type=jax.ShapeDtypeStruct((n_rows, dim), jnp.float32),
        scratch_types=[
            pltpu.VMEM((n_ch, _CH), jnp.int32),
            pltpu.VMEM((_CH, dim), jnp.float32),
            pltpu.SemaphoreType.DMA,
        ],
    )
    def gather_kernel(table_hbm, idx_hbm, out_hbm, idx_v, buf, sem):
        wid = lax.axis_index("s") * nc + lax.axis_index("c")
        base = wid * b_per_w
        pltpu.sync_copy(idx_hbm.at[wid], idx_v)

        def step(j, carry):
            pltpu.async_copy(table_hbm.at[idx_v.at[j]], buf, sem).wait()
            pltpu.sync_copy(buf, out_hbm.at[pl.ds(base + j * _CH, _CH)])
            return carry

        lax.fori_loop(0, n_ch, step, 0)

    def run(table, idx3):
        return gather_kernel(table, idx3)

    return run, nw, n_ch


def kernel(x, emb_wi):
    b, s = x.shape
    v, d = emb_wi.shape
    n = b * s
    run, nw, n_ch = _build_gather(n, v, d)
    idx3 = x.astype(jnp.int32).reshape(nw, n_ch, _CH)
    out = run(emb_wi, idx3)
    return out.reshape(b, s, d)


# speedup vs baseline: 4.6567x; 1.1397x over previous
import functools

import jax
import jax.numpy as jnp
from jax import lax
from jax.experimental import pallas as pl
from jax.experimental.pallas import tpu as pltpu
from jax.experimental.pallas import tpu_sc as plsc

_CH = 128
_NBUF = 10


@functools.lru_cache(maxsize=None)
def _build_gather(n_rows: int, vocab: int, dim: int):
    info = plsc.get_sparse_core_info()
    nc, ns = info.num_cores, info.num_subcores
    nw = nc * ns
    assert n_rows % (nw * _CH) == 0
    b_per_w = n_rows // nw
    n_ch = b_per_w // _CH
    nbuf = _NBUF
    assert n_ch % nbuf == 0
    rounds = n_ch // nbuf

    mesh = plsc.VectorSubcoreMesh(core_axis_name="c", subcore_axis_name="s")

    @functools.partial(
        pl.kernel,
        mesh=mesh,
        compiler_params=pltpu.CompilerParams(use_tc_tiling_on_sc=False),
        out_type=jax.ShapeDtypeStruct((n_rows, dim), jnp.float32),
        scratch_types=[
            pltpu.VMEM((n_ch, _CH), jnp.int32),
            pltpu.VMEM((nbuf, _CH, dim), jnp.float32),
        ]
        + [pltpu.SemaphoreType.DMA] * (2 * nbuf),
    )
    def gather_kernel(table_hbm, idx_hbm, out_hbm, idx_v, bufs, *sems):
        sem_g = sems[:nbuf]
        sem_s = sems[nbuf:]
        wid = lax.axis_index("s") * nc + lax.axis_index("c")
        base = wid * b_per_w
        pltpu.sync_copy(idx_hbm.at[wid], idx_v)

        def gather_start(j, b):
            pltpu.async_copy(table_hbm.at[idx_v.at[j]], bufs.at[b], sem_g[b])

        def gather_wait(j, b):
            pltpu.make_async_copy(
                table_hbm.at[idx_v.at[j]], bufs.at[b], sem_g[b]
            ).wait()

        def scatter_start(j, b):
            return pltpu.async_copy(
                bufs.at[b], out_hbm.at[pl.ds(base + j * _CH, _CH)], sem_s[b]
            )

        for b in range(nbuf):
            gather_start(b, b)

        def step(r, carry):
            scatters = []
            for b in range(nbuf):
                j = r * nbuf + b
                gather_wait(j, b)
                scatters.append(scatter_start(j, b))
            for b in range(nbuf):
                j = r * nbuf + b
                scatters[b].wait()
                gather_start(j + nbuf, b)
            return carry

        lax.fori_loop(0, rounds - 1, step, 0)

        scatters = []
        for b in range(nbuf):
            j = (rounds - 1) * nbuf + b
            gather_wait(j, b)
            scatters.append(scatter_start(j, b))
        for h in scatters:
            h.wait()

    def run(table, idx3):
        return gather_kernel(table, idx3)

    return run, nw, n_ch


def kernel(x, emb_wi):
    b, s = x.shape
    v, d = emb_wi.shape
    n = b * s
    run, nw, n_ch = _build_gather(n, v, d)
    idx3 = x.astype(jnp.int32).reshape(nw, n_ch, _CH)
    out = run(emb_wi, idx3)
    return out.reshape(b, s, d)
